# baseline (speedup 1.0000x reference)
import jax
import jax.numpy as jnp
from jax import lax
from jax.experimental import pallas as pl
from jax.experimental.pallas import tpu as pltpu
from jax.experimental.pallas import tpu_sc as plsc

B = 16384
F = 128
NC = 2
NS = 16
NW = NC * NS
BPW = B // NW
CHUNK = 64
NCHUNK = BPW // CHUNK
GROUPS = CHUNK // 16


def _sc_body(users, items, uw, iw, ub, ib, out,
             uidx, iidx, urows0, urows1, irows0, irows1,
             ubv0, ubv1, ibv0, ibv1, outv, sems):
    wid = lax.axis_index("s") * NC + lax.axis_index("c")
    base = wid * BPW

    urows = (urows0, urows1)
    irows = (irows0, irows1)
    ubv = (ubv0, ubv1)
    ibv = (ibv0, ibv1)

    hu = pltpu.async_copy(users.at[pl.ds(base, BPW)], uidx, sems.at[0])
    hi = pltpu.async_copy(items.at[pl.ds(base, BPW)], iidx, sems.at[1])
    hu.wait()
    hi.wait()

    def issue(c, slot):
        uc = uidx.at[pl.ds(c * CHUNK, CHUNK)]
        ic = iidx.at[pl.ds(c * CHUNK, CHUNK)]
        return [
            pltpu.async_copy(uw.at[uc], urows[slot], sems.at[slot]),
            pltpu.async_copy(iw.at[ic], irows[slot], sems.at[slot]),
            pltpu.async_copy(ub.at[uc], ubv[slot], sems.at[slot]),
            pltpu.async_copy(ib.at[ic], ibv[slot], sems.at[slot]),
        ]

    def compute(c, slot):
        ur = urows[slot]
        ir = irows[slot]
        ubc = ubv[slot]
        ibc = ibv[slot]

        lane = lax.iota(jnp.int32, 16)

        def gbody(g, carry):
            def tbody(t, sums):
                for j in range(8):
                    r = g * 16 + t * 8 + j
                    parts = [ur[r, pl.ds(k * 16, 16)]
                             * ir[r, pl.ds(k * 16, 16)]
                             for k in range(F // 16)]
                    while len(parts) > 1:
                        parts = [parts[i] + parts[i + 1]
                                 for i in range(0, len(parts), 2)]
                    s = jnp.sum(parts[0])
                    sums = jnp.where(lane == t * 8 + j,
                                     jnp.full((16,), s), sums)
                return sums

            sums = lax.fori_loop(0, 2, tbody, jnp.zeros((16,), jnp.float32))
            res = sums + ubc[pl.ds(g * 16, 16)] + ibc[pl.ds(g * 16, 16)]
            y = 5.0 / (1.0 + jnp.exp(-res))
            outv[pl.ds(c * CHUNK + g * 16, 16)] = y
            return carry

        lax.fori_loop(0, GROUPS, gbody, 0)

    handles = issue(0, 0)
    for c in range(NCHUNK):
        slot = c % 2
        for h in handles:
            h.wait()
        if c + 1 < NCHUNK:
            handles = issue(c + 1, 1 - slot)
        compute(c, slot)

    pltpu.sync_copy(outv, out.at[pl.ds(base, BPW)])


@jax.jit
def kernel(users, items, u_weight, i_weight, u_bias, i_bias):
    mesh = plsc.VectorSubcoreMesh(core_axis_name="c", subcore_axis_name="s")
    run = pl.kernel(
        _sc_body,
        out_type=jax.ShapeDtypeStruct((B,), jnp.float32),
        mesh=mesh,
        compiler_params=pltpu.CompilerParams(needs_layout_passes=False),
        scratch_types=[
            pltpu.VMEM((BPW,), jnp.int32),
            pltpu.VMEM((BPW,), jnp.int32),
            pltpu.VMEM((CHUNK, F), jnp.float32),
            pltpu.VMEM((CHUNK, F), jnp.float32),
            pltpu.VMEM((CHUNK, F), jnp.float32),
            pltpu.VMEM((CHUNK, F), jnp.float32),
            pltpu.VMEM((CHUNK,), jnp.float32),
            pltpu.VMEM((CHUNK,), jnp.float32),
            pltpu.VMEM((CHUNK,), jnp.float32),
            pltpu.VMEM((CHUNK,), jnp.float32),
            pltpu.VMEM((BPW,), jnp.float32),
            pltpu.SemaphoreType.DMA((2,)),
        ],
    )
    return run(users.astype(jnp.int32), items.astype(jnp.int32),
               u_weight, i_weight, u_bias.reshape(-1), i_bias.reshape(-1))


# speedup vs baseline: 1.0905x; 1.0905x over previous
import jax
import jax.numpy as jnp
from jax import lax
from jax.experimental import pallas as pl
from jax.experimental.pallas import tpu as pltpu
from jax.experimental.pallas import tpu_sc as plsc

B = 16384
F = 128
NC = 2
NS = 16
NW = NC * NS
BPW = B // NW
CHUNK = 128
NCHUNK = BPW // CHUNK
GROUPS = CHUNK // 16


def _sc_body(users, items, uw, iw, ub, ib, out,
             uidx, iidx, urows0, urows1, irows0, irows1,
             ubv0, ubv1, ibv0, ibv1, outv, sems):
    wid = lax.axis_index("s") * NC + lax.axis_index("c")
    base = wid * BPW

    urows = (urows0, urows1)
    irows = (irows0, irows1)
    ubv = (ubv0, ubv1)
    ibv = (ibv0, ibv1)

    hu = pltpu.async_copy(users.at[pl.ds(base, BPW)], uidx, sems.at[0])
    hi = pltpu.async_copy(items.at[pl.ds(base, BPW)], iidx, sems.at[1])
    hu.wait()
    hi.wait()

    def issue(c, slot):
        uc = uidx.at[pl.ds(c * CHUNK, CHUNK)]
        ic = iidx.at[pl.ds(c * CHUNK, CHUNK)]
        return [
            pltpu.async_copy(uw.at[uc], urows[slot], sems.at[slot]),
            pltpu.async_copy(iw.at[ic], irows[slot], sems.at[slot]),
            pltpu.async_copy(ub.at[uc], ubv[slot], sems.at[slot]),
            pltpu.async_copy(ib.at[ic], ibv[slot], sems.at[slot]),
        ]

    def compute(c, slot):
        ur = urows[slot]
        ir = irows[slot]
        ubc = ubv[slot]
        ibc = ibv[slot]

        lane = lax.iota(jnp.int32, 16)

        def gbody(g, carry):
            def tbody(t, sums):
                for j in range(8):
                    r = g * 16 + t * 8 + j
                    acc = ur[r, pl.ds(0, 16)] * ir[r, pl.ds(0, 16)]
                    for k in range(1, F // 16):
                        acc = acc + (ur[r, pl.ds(k * 16, 16)]
                                     * ir[r, pl.ds(k * 16, 16)])
                    s = jnp.sum(acc)
                    sums = jnp.where(lane == t * 8 + j,
                                     jnp.full((16,), s), sums)
                return sums

            sums = lax.fori_loop(0, 2, tbody, jnp.zeros((16,), jnp.float32))
            res = sums + ubc[pl.ds(g * 16, 16)] + ibc[pl.ds(g * 16, 16)]
            y = 5.0 / (1.0 + jnp.exp(-res))
            outv[pl.ds(c * CHUNK + g * 16, 16)] = y
            return carry

        lax.fori_loop(0, GROUPS, gbody, 0)

    handles = issue(0, 0)
    for c in range(NCHUNK):
        slot = c % 2
        for h in handles:
            h.wait()
        if c + 1 < NCHUNK:
            handles = issue(c + 1, 1 - slot)
        compute(c, slot)

    pltpu.sync_copy(outv, out.at[pl.ds(base, BPW)])


@jax.jit
def kernel(users, items, u_weight, i_weight, u_bias, i_bias):
    mesh = plsc.VectorSubcoreMesh(core_axis_name="c", subcore_axis_name="s")
    run = pl.kernel(
        _sc_body,
        out_type=jax.ShapeDtypeStruct((B,), jnp.float32),
        mesh=mesh,
        compiler_params=pltpu.CompilerParams(needs_layout_passes=False),
        scratch_types=[
            pltpu.VMEM((BPW,), jnp.int32),
            pltpu.VMEM((BPW,), jnp.int32),
            pltpu.VMEM((CHUNK, F), jnp.float32),
            pltpu.VMEM((CHUNK, F), jnp.float32),
            pltpu.VMEM((CHUNK, F), jnp.float32),
            pltpu.VMEM((CHUNK, F), jnp.float32),
            pltpu.VMEM((CHUNK,), jnp.float32),
            pltpu.VMEM((CHUNK,), jnp.float32),
            pltpu.VMEM((CHUNK,), jnp.float32),
            pltpu.VMEM((CHUNK,), jnp.float32),
            pltpu.VMEM((BPW,), jnp.float32),
            pltpu.SemaphoreType.DMA((2,)),
        ],
    )
    return run(users.astype(jnp.int32), items.astype(jnp.int32),
               u_weight, i_weight, u_bias.reshape(-1), i_bias.reshape(-1))
